# baseline (speedup 1.0000x reference)
import functools

import jax
import jax.numpy as jnp
from jax import lax
from jax.experimental import pallas as pl
from jax.experimental.pallas import tpu as pltpu
from jax.experimental.pallas import tpu_sc as plsc

N = 10000
E = 320000
D = 128
CHUNK = 128
NCHUNKS = E // CHUNK
NTILES = 32
ITERS = -(-NCHUNKS // NTILES)
NB = 1
ROUNDS = -(-ITERS // NB)
ROWS = 624
TAIL = N - 16 * ROWS

_MESH = plsc.VectorSubcoreMesh(core_axis_name="c", subcore_axis_name="s")


def _deg_body(dst_hbm, ones_hbm, z_hbm, deg_hbm, idx_v, ones_v, deg_sh):
    c = lax.axis_index("c")
    s = lax.axis_index("s")
    wid = c * 16 + s
    off = pl.multiple_of(s * ROWS, 8)
    pltpu.sync_copy(ones_hbm, ones_v)
    pltpu.sync_copy(z_hbm, deg_sh.at[pl.ds(off, ROWS)])

    @pl.when(s == 0)
    def _():
        pltpu.sync_copy(z_hbm.at[pl.ds(0, TAIL)],
                        deg_sh.at[pl.ds(16 * ROWS, TAIL)])

    plsc.subcore_barrier()

    def body(j, carry):
        cid = wid + j * NTILES

        @pl.when(cid < NCHUNKS)
        def _():
            base = pl.multiple_of(cid * CHUNK, CHUNK)
            pltpu.sync_copy(dst_hbm.at[pl.ds(base, CHUNK)], idx_v)
            pltpu.sync_copy(ones_v, deg_sh.at[idx_v], add=True)

        return carry

    lax.fori_loop(0, ITERS, body, 0)
    plsc.subcore_barrier()
    pltpu.sync_copy(deg_sh.at[pl.ds(off, ROWS)],
                    deg_hbm.at[c, pl.ds(off, ROWS)])

    @pl.when(s == 0)
    def _():
        pltpu.sync_copy(deg_sh.at[pl.ds(16 * ROWS, TAIL)],
                        deg_hbm.at[c, pl.ds(16 * ROWS, TAIL)])


@functools.partial(
    pl.kernel,
    out_type=jax.ShapeDtypeStruct((2, N, D), jnp.float32),
    mesh=_MESH,
    scratch_types=[
        pltpu.VMEM((CHUNK,), jnp.int32),
        pltpu.VMEM((CHUNK, D), jnp.float32),
        pltpu.VMEM_SHARED((N, D), jnp.float32),
    ],
)
def _deg_call(dst_hbm, ones_hbm, z_hbm, deg_hbm, idx_v, ones_v, deg_sh):
    _deg_body(dst_hbm, ones_hbm, z_hbm, deg_hbm, idx_v, ones_v, deg_sh)


def _prop_body(hp_hbm, src_hbm, dst_hbm, z_hbm, out_hbm,
               idx_s, idx_d, rows_v, out_sh, sem_g):
    c = lax.axis_index("c")
    s = lax.axis_index("s")
    wid = c * 16 + s
    off = pl.multiple_of(s * ROWS, 8)
    pltpu.sync_copy(z_hbm, out_sh.at[pl.ds(off, ROWS)])

    @pl.when(s == 0)
    def _():
        pltpu.sync_copy(z_hbm.at[pl.ds(0, TAIL)],
                        out_sh.at[pl.ds(16 * ROWS, TAIL)])

    plsc.subcore_barrier()

    def body(j, carry):
        cid = wid + j * NTILES

        @pl.when(cid < NCHUNKS)
        def _():
            base = pl.multiple_of(cid * CHUNK, CHUNK)
            pltpu.sync_copy(src_hbm.at[pl.ds(base, CHUNK)], idx_s[0])
            pltpu.sync_copy(dst_hbm.at[pl.ds(base, CHUNK)], idx_d[0])
            pltpu.async_copy(hp_hbm.at[idx_s[0]], rows_v[0], sem_g).wait()
            pltpu.sync_copy(rows_v[0], out_sh.at[idx_d[0]], add=True)

        return carry

    lax.fori_loop(0, ITERS, body, 0)
    plsc.subcore_barrier()
    pltpu.sync_copy(out_sh.at[pl.ds(off, ROWS)],
                    out_hbm.at[c, pl.ds(off, ROWS)])

    @pl.when(s == 0)
    def _():
        pltpu.sync_copy(out_sh.at[pl.ds(16 * ROWS, TAIL)],
                        out_hbm.at[c, pl.ds(16 * ROWS, TAIL)])


@functools.partial(
    pl.kernel,
    out_type=jax.ShapeDtypeStruct((2, N, D), jnp.float32),
    mesh=_MESH,
    scratch_types=(
        [pltpu.VMEM((CHUNK,), jnp.int32) for _ in range(NB)]
        + [pltpu.VMEM((CHUNK,), jnp.int32) for _ in range(NB)]
        + [pltpu.VMEM((CHUNK, D), jnp.float32) for _ in range(NB)]
        + [
            pltpu.VMEM_SHARED((N, D), jnp.float32),
            pltpu.SemaphoreType.DMA,
        ]
    ),
)
def _prop_call(hp_hbm, src_hbm, dst_hbm, z_hbm, out_hbm, *scr):
    idx_s = list(scr[0:NB])
    idx_d = list(scr[NB:2 * NB])
    rows_v = list(scr[2 * NB:3 * NB])
    out_sh, sem_g = scr[3 * NB:]
    _prop_body(hp_hbm, src_hbm, dst_hbm, z_hbm, out_hbm,
               idx_s, idx_d, rows_v, out_sh, sem_g)


def _dinv(d0_ref, d1_ref):
    deg = d0_ref[:, 0:1] + d1_ref[:, 0:1] + 1.0
    return lax.rsqrt(deg)


def _tca_body(x_ref, w_ref, d0_ref, d1_ref, o_ref):
    dinv = _dinv(d0_ref, d1_ref)
    o_ref[...] = jnp.dot(x_ref[...], w_ref[...],
                         preferred_element_type=jnp.float32) * dinv


def _bn(s0_ref, s1_ref, hp_ref, dinv, g_ref, bt_ref):
    u = (s0_ref[...] + s1_ref[...] + hp_ref[...]) * dinv
    m = jnp.mean(u, axis=0, keepdims=True)
    d = u - m
    v = jnp.mean(d * d, axis=0, keepdims=True)
    return d * lax.rsqrt(v + 1e-5) * g_ref[...] + bt_ref[...]


def _tcb_body(s0_ref, s1_ref, hp_ref, d0_ref, d1_ref, g_ref, bt_ref, wn_ref,
              o_ref):
    dinv = _dinv(d0_ref, d1_ref)
    y = _bn(s0_ref, s1_ref, hp_ref, dinv, g_ref, bt_ref)
    o_ref[...] = jnp.dot(y, wn_ref[...],
                         preferred_element_type=jnp.float32) * dinv


def _tcc_body(s0_ref, s1_ref, hp_ref, d0_ref, d1_ref, g_ref, bt_ref, o_ref):
    dinv = _dinv(d0_ref, d1_ref)
    y = _bn(s0_ref, s1_ref, hp_ref, dinv, g_ref, bt_ref)
    o_ref[...] = y[:1024, :]


_tca = pl.pallas_call(
    _tca_body, out_shape=jax.ShapeDtypeStruct((N, D), jnp.float32))
_tcb = pl.pallas_call(
    _tcb_body, out_shape=jax.ShapeDtypeStruct((N, D), jnp.float32))
_tcc = pl.pallas_call(
    _tcc_body, out_shape=jax.ShapeDtypeStruct((1024, D), jnp.float32))


def kernel(x, edge_index, W1, b1, W2, b2, W3, b3, g1, bt1, g2, bt2, g3, bt3):
    src = edge_index[0]
    dst = edge_index[1]
    ones_blk = jnp.ones((CHUNK, D), jnp.float32)
    z128 = jnp.zeros((ROWS, D), jnp.float32)

    deg = _deg_call(dst, ones_blk, z128)
    d0 = deg[0]
    d1 = deg[1]

    hp1 = _tca(x, W1, d0, d1)
    s1 = _prop_call(hp1, src, dst, z128)
    hp2 = _tcb(s1[0], s1[1], hp1, d0, d1,
               g1.reshape(1, D), bt1.reshape(1, D), W2)
    s2 = _prop_call(hp2, src, dst, z128)
    hp3 = _tcb(s2[0], s2[1], hp2, d0, d1,
               g2.reshape(1, D), bt2.reshape(1, D), W3)
    s3 = _prop_call(hp3, src, dst, z128)
    return _tcc(s3[0], s3[1], hp3, d0, d1,
                g3.reshape(1, D), bt3.reshape(1, D))


# speedup vs baseline: 1.5843x; 1.5843x over previous
import functools

import jax
import jax.numpy as jnp
from jax import lax
from jax.experimental import pallas as pl
from jax.experimental.pallas import tpu as pltpu
from jax.experimental.pallas import tpu_sc as plsc

N = 10000
E = 320000
D = 128
CHUNK = 128
NCHUNKS = E // CHUNK
NTILES = 32
ITERS = -(-NCHUNKS // NTILES)
PIPE_ROUNDS = -(-(ITERS + 2) // 6)
ROWS = 624
TAIL = N - 16 * ROWS

_MESH = plsc.VectorSubcoreMesh(core_axis_name="c", subcore_axis_name="s")


def _deg_body(dst_hbm, ones_hbm, z_hbm, deg_hbm, idx_v, ones_v, deg_sh):
    c = lax.axis_index("c")
    s = lax.axis_index("s")
    wid = c * 16 + s
    off = pl.multiple_of(s * ROWS, 8)
    pltpu.sync_copy(ones_hbm, ones_v)
    pltpu.sync_copy(z_hbm, deg_sh.at[pl.ds(off, ROWS)])

    @pl.when(s == 0)
    def _():
        pltpu.sync_copy(z_hbm.at[pl.ds(0, TAIL)],
                        deg_sh.at[pl.ds(16 * ROWS, TAIL)])

    plsc.subcore_barrier()

    def body(j, carry):
        cid = wid + j * NTILES

        @pl.when(cid < NCHUNKS)
        def _():
            base = pl.multiple_of(cid * CHUNK, CHUNK)
            pltpu.sync_copy(dst_hbm.at[pl.ds(base, CHUNK)], idx_v)
            pltpu.sync_copy(ones_v, deg_sh.at[idx_v], add=True)

        return carry

    lax.fori_loop(0, ITERS, body, 0)
    plsc.subcore_barrier()
    pltpu.sync_copy(deg_sh.at[pl.ds(off, ROWS)],
                    deg_hbm.at[c, pl.ds(off, ROWS)])

    @pl.when(s == 0)
    def _():
        pltpu.sync_copy(deg_sh.at[pl.ds(16 * ROWS, TAIL)],
                        deg_hbm.at[c, pl.ds(16 * ROWS, TAIL)])


@functools.partial(
    pl.kernel,
    out_type=jax.ShapeDtypeStruct((2, N, D), jnp.float32),
    mesh=_MESH,
    scratch_types=[
        pltpu.VMEM((CHUNK,), jnp.int32),
        pltpu.VMEM((CHUNK, D), jnp.float32),
        pltpu.VMEM_SHARED((N, D), jnp.float32),
    ],
)
def _deg_call(dst_hbm, ones_hbm, z_hbm, deg_hbm, idx_v, ones_v, deg_sh):
    _deg_body(dst_hbm, ones_hbm, z_hbm, deg_hbm, idx_v, ones_v, deg_sh)


def _prop_body(hp_hbm, src_hbm, dst_hbm, z_hbm, out_hbm,
               idx_s, idx_d, rows_v, out_sh, sem_i, sem_g, sem_s):
    c = lax.axis_index("c")
    s = lax.axis_index("s")
    wid = c * 16 + s
    off = pl.multiple_of(s * ROWS, 8)
    pltpu.sync_copy(z_hbm, out_sh.at[pl.ds(off, ROWS)])

    @pl.when(s == 0)
    def _():
        pltpu.sync_copy(z_hbm.at[pl.ds(0, TAIL)],
                        out_sh.at[pl.ds(16 * ROWS, TAIL)])

    plsc.subcore_barrier()

    def valid(k):
        return (wid + k * NTILES) < NCHUNKS

    def idx_descs(k, ib):
        base = pl.multiple_of((wid + k * NTILES) * CHUNK, CHUNK)
        return (
            pltpu.make_async_copy(src_hbm.at[pl.ds(base, CHUNK)],
                                  idx_s[ib], sem_i),
            pltpu.make_async_copy(dst_hbm.at[pl.ds(base, CHUNK)],
                                  idx_d[ib], sem_i),
        )

    def g_desc(rb, ib):
        return pltpu.make_async_copy(hp_hbm.at[idx_s[ib]], rows_v[rb], sem_g)

    def s_desc(rb, ib):
        return pltpu.make_async_copy(rows_v[rb], out_sh.at[idx_d[ib]],
                                     sem_s[rb])

    for d in idx_descs(0, 0):
        d.start()

    def body(r, carry):
        for b in range(6):
            j = r * 6 + b

            @pl.when(valid(j))
            def _(j=j, b=b):
                for d in idx_descs(j, b % 3):
                    d.wait()

            pm1 = valid(j - 1) if b >= 1 else ((r >= 1) & valid(j - 1))

            @pl.when(pm1)
            def _(j=j, b=b):
                g_desc((b - 1) % 2, (b - 1) % 3).wait()
                s_desc((b - 1) % 2, (b - 1) % 3).start(add=True)

            pm2 = valid(j - 2) if b >= 2 else ((r >= 1) & valid(j - 2))

            @pl.when(pm2)
            def _(j=j, b=b):
                s_desc((b - 2) % 2, (b - 2) % 3).wait()

            @pl.when(valid(j))
            def _(j=j, b=b):
                g_desc(b % 2, b % 3).start()

            @pl.when(valid(j + 1))
            def _(j=j, b=b):
                for d in idx_descs(j + 1, (b + 1) % 3):
                    d.start()

        return carry

    lax.fori_loop(0, PIPE_ROUNDS, body, 0)
    plsc.subcore_barrier()
    pltpu.sync_copy(out_sh.at[pl.ds(off, ROWS)],
                    out_hbm.at[c, pl.ds(off, ROWS)])

    @pl.when(s == 0)
    def _():
        pltpu.sync_copy(out_sh.at[pl.ds(16 * ROWS, TAIL)],
                        out_hbm.at[c, pl.ds(16 * ROWS, TAIL)])


@functools.partial(
    pl.kernel,
    out_type=jax.ShapeDtypeStruct((2, N, D), jnp.float32),
    mesh=_MESH,
    scratch_types=(
        [pltpu.VMEM((CHUNK,), jnp.int32) for _ in range(3)]
        + [pltpu.VMEM((CHUNK,), jnp.int32) for _ in range(3)]
        + [pltpu.VMEM((CHUNK, D), jnp.float32) for _ in range(2)]
        + [
            pltpu.VMEM_SHARED((N, D), jnp.float32),
            pltpu.SemaphoreType.DMA,
            pltpu.SemaphoreType.DMA,
            pltpu.SemaphoreType.DMA,
            pltpu.SemaphoreType.DMA,
        ]
    ),
)
def _prop_call(hp_hbm, src_hbm, dst_hbm, z_hbm, out_hbm, *scr):
    idx_s = list(scr[0:3])
    idx_d = list(scr[3:6])
    rows_v = list(scr[6:8])
    out_sh, sem_i, sem_g, sem_s0, sem_s1 = scr[8:]
    _prop_body(hp_hbm, src_hbm, dst_hbm, z_hbm, out_hbm,
               idx_s, idx_d, rows_v, out_sh, sem_i, sem_g,
               [sem_s0, sem_s1])


def _dinv(d0_ref, d1_ref):
    deg = d0_ref[:, 0:1] + d1_ref[:, 0:1] + 1.0
    return lax.rsqrt(deg)


def _tca_body(x_ref, w_ref, d0_ref, d1_ref, o_ref):
    dinv = _dinv(d0_ref, d1_ref)
    o_ref[...] = jnp.dot(x_ref[...], w_ref[...],
                         preferred_element_type=jnp.float32) * dinv


def _bn(s0_ref, s1_ref, hp_ref, dinv, g_ref, bt_ref):
    u = (s0_ref[...] + s1_ref[...] + hp_ref[...]) * dinv
    m = jnp.mean(u, axis=0, keepdims=True)
    d = u - m
    v = jnp.mean(d * d, axis=0, keepdims=True)
    return d * lax.rsqrt(v + 1e-5) * g_ref[...] + bt_ref[...]


def _tcb_body(s0_ref, s1_ref, hp_ref, d0_ref, d1_ref, g_ref, bt_ref, wn_ref,
              o_ref):
    dinv = _dinv(d0_ref, d1_ref)
    y = _bn(s0_ref, s1_ref, hp_ref, dinv, g_ref, bt_ref)
    o_ref[...] = jnp.dot(y, wn_ref[...],
                         preferred_element_type=jnp.float32) * dinv


def _tcc_body(s0_ref, s1_ref, hp_ref, d0_ref, d1_ref, g_ref, bt_ref, o_ref):
    dinv = _dinv(d0_ref, d1_ref)
    y = _bn(s0_ref, s1_ref, hp_ref, dinv, g_ref, bt_ref)
    o_ref[...] = y[:1024, :]


_tca = pl.pallas_call(
    _tca_body, out_shape=jax.ShapeDtypeStruct((N, D), jnp.float32))
_tcb = pl.pallas_call(
    _tcb_body, out_shape=jax.ShapeDtypeStruct((N, D), jnp.float32))
_tcc = pl.pallas_call(
    _tcc_body, out_shape=jax.ShapeDtypeStruct((1024, D), jnp.float32))


def kernel(x, edge_index, W1, b1, W2, b2, W3, b3, g1, bt1, g2, bt2, g3, bt3):
    src = edge_index[0]
    dst = edge_index[1]
    ones_blk = jnp.ones((CHUNK, D), jnp.float32)
    z128 = jnp.zeros((ROWS, D), jnp.float32)

    deg = _deg_call(dst, ones_blk, z128)
    d0 = deg[0]
    d1 = deg[1]

    hp1 = _tca(x, W1, d0, d1)
    s1 = _prop_call(hp1, src, dst, z128)
    hp2 = _tcb(s1[0], s1[1], hp1, d0, d1,
               g1.reshape(1, D), bt1.reshape(1, D), W2)
    s2 = _prop_call(hp2, src, dst, z128)
    hp3 = _tcb(s2[0], s2[1], hp2, d0, d1,
               g2.reshape(1, D), bt2.reshape(1, D), W3)
    s3 = _prop_call(hp3, src, dst, z128)
    return _tcc(s3[0], s3[1], hp3, d0, d1,
                g3.reshape(1, D), bt3.reshape(1, D))


# speedup vs baseline: 1.6660x; 1.0516x over previous
import functools

import jax
import jax.numpy as jnp
from jax import lax
from jax.experimental import pallas as pl
from jax.experimental.pallas import tpu as pltpu
from jax.experimental.pallas import tpu_sc as plsc

N = 10000
E = 320000
D = 128
CHUNK = 128
NCHUNKS = E // CHUNK
NTILES = 32
ITERS = -(-NCHUNKS // NTILES)
PIPE_ROUNDS = -(-(ITERS + 2) // 6)
ROWS = 624
TAIL = N - 16 * ROWS

_MESH = plsc.VectorSubcoreMesh(core_axis_name="c", subcore_axis_name="s")


def _deg_body(dst_hbm, ones_hbm, z_hbm, deg_hbm, idx_v, ones_v, deg_sh,
              sem_i, sem_s):
    c = lax.axis_index("c")
    s = lax.axis_index("s")
    wid = c * 16 + s
    off = pl.multiple_of(s * ROWS, 8)
    pltpu.sync_copy(ones_hbm, ones_v)
    pltpu.sync_copy(z_hbm, deg_sh.at[pl.ds(off, ROWS)])

    @pl.when(s == 0)
    def _():
        pltpu.sync_copy(z_hbm.at[pl.ds(0, TAIL)],
                        deg_sh.at[pl.ds(16 * ROWS, TAIL)])

    plsc.subcore_barrier()

    def valid(k):
        return (wid + k * NTILES) < NCHUNKS

    def i_desc(k, ib):
        base = pl.multiple_of((wid + k * NTILES) * CHUNK, CHUNK)
        return pltpu.make_async_copy(dst_hbm.at[pl.ds(base, CHUNK)],
                                     idx_v[ib], sem_i)

    def s_desc(sb, ib):
        return pltpu.make_async_copy(ones_v, deg_sh.at[idx_v[ib]],
                                     sem_s[sb])

    i_desc(0, 0).start()

    def body(r, carry):
        for b in range(6):
            j = r * 6 + b

            @pl.when(valid(j))
            def _(j=j, b=b):
                i_desc(j, b % 3).wait()

            pm2 = valid(j - 2) if b >= 2 else ((r >= 1) & valid(j - 2))

            @pl.when(pm2)
            def _(j=j, b=b):
                s_desc((b - 2) % 2, (b - 2) % 3).wait()

            @pl.when(valid(j))
            def _(j=j, b=b):
                s_desc(b % 2, b % 3).start(add=True)

            @pl.when(valid(j + 1))
            def _(j=j, b=b):
                i_desc(j + 1, (b + 1) % 3).start()

        return carry

    lax.fori_loop(0, PIPE_ROUNDS, body, 0)
    plsc.subcore_barrier()
    pltpu.sync_copy(deg_sh.at[pl.ds(off, ROWS)],
                    deg_hbm.at[c, pl.ds(off, ROWS)])

    @pl.when(s == 0)
    def _():
        pltpu.sync_copy(deg_sh.at[pl.ds(16 * ROWS, TAIL)],
                        deg_hbm.at[c, pl.ds(16 * ROWS, TAIL)])


@functools.partial(
    pl.kernel,
    out_type=jax.ShapeDtypeStruct((2, N, D), jnp.float32),
    mesh=_MESH,
    scratch_types=[
        pltpu.VMEM((CHUNK,), jnp.int32),
        pltpu.VMEM((CHUNK,), jnp.int32),
        pltpu.VMEM((CHUNK,), jnp.int32),
        pltpu.VMEM((CHUNK, D), jnp.float32),
        pltpu.VMEM_SHARED((N, D), jnp.float32),
        pltpu.SemaphoreType.DMA,
        pltpu.SemaphoreType.DMA,
        pltpu.SemaphoreType.DMA,
    ],
)
def _deg_call(dst_hbm, ones_hbm, z_hbm, deg_hbm, i0, i1, i2, ones_v, deg_sh,
              sem_i, sem_s0, sem_s1):
    _deg_body(dst_hbm, ones_hbm, z_hbm, deg_hbm, [i0, i1, i2], ones_v,
              deg_sh, sem_i, [sem_s0, sem_s1])


def _prop_body(hp_hbm, src_hbm, dst_hbm, z_hbm, out_hbm,
               idx_s, idx_d, rows_v, out_sh, sem_i, sem_g, sem_s):
    c = lax.axis_index("c")
    s = lax.axis_index("s")
    wid = c * 16 + s
    off = pl.multiple_of(s * ROWS, 8)
    pltpu.sync_copy(z_hbm, out_sh.at[pl.ds(off, ROWS)])

    @pl.when(s == 0)
    def _():
        pltpu.sync_copy(z_hbm.at[pl.ds(0, TAIL)],
                        out_sh.at[pl.ds(16 * ROWS, TAIL)])

    plsc.subcore_barrier()

    def valid(k):
        return (wid + k * NTILES) < NCHUNKS

    def idx_descs(k, ib):
        base = pl.multiple_of((wid + k * NTILES) * CHUNK, CHUNK)
        return (
            pltpu.make_async_copy(src_hbm.at[pl.ds(base, CHUNK)],
                                  idx_s[ib], sem_i),
            pltpu.make_async_copy(dst_hbm.at[pl.ds(base, CHUNK)],
                                  idx_d[ib], sem_i),
        )

    def g_desc(rb, ib):
        return pltpu.make_async_copy(hp_hbm.at[idx_s[ib]], rows_v[rb], sem_g)

    def s_desc(rb, ib):
        return pltpu.make_async_copy(rows_v[rb], out_sh.at[idx_d[ib]],
                                     sem_s[rb])

    for d in idx_descs(0, 0):
        d.start()

    def body(r, carry):
        for b in range(6):
            j = r * 6 + b

            @pl.when(valid(j))
            def _(j=j, b=b):
                for d in idx_descs(j, b % 3):
                    d.wait()

            pm1 = valid(j - 1) if b >= 1 else ((r >= 1) & valid(j - 1))

            @pl.when(pm1)
            def _(j=j, b=b):
                g_desc((b - 1) % 2, (b - 1) % 3).wait()
                s_desc((b - 1) % 2, (b - 1) % 3).start(add=True)

            pm2 = valid(j - 2) if b >= 2 else ((r >= 1) & valid(j - 2))

            @pl.when(pm2)
            def _(j=j, b=b):
                s_desc((b - 2) % 2, (b - 2) % 3).wait()

            @pl.when(valid(j))
            def _(j=j, b=b):
                g_desc(b % 2, b % 3).start()

            @pl.when(valid(j + 1))
            def _(j=j, b=b):
                for d in idx_descs(j + 1, (b + 1) % 3):
                    d.start()

        return carry

    lax.fori_loop(0, PIPE_ROUNDS, body, 0)
    plsc.subcore_barrier()
    pltpu.sync_copy(out_sh.at[pl.ds(off, ROWS)],
                    out_hbm.at[c, pl.ds(off, ROWS)])

    @pl.when(s == 0)
    def _():
        pltpu.sync_copy(out_sh.at[pl.ds(16 * ROWS, TAIL)],
                        out_hbm.at[c, pl.ds(16 * ROWS, TAIL)])


@functools.partial(
    pl.kernel,
    out_type=jax.ShapeDtypeStruct((2, N, D), jnp.float32),
    mesh=_MESH,
    scratch_types=(
        [pltpu.VMEM((CHUNK,), jnp.int32) for _ in range(3)]
        + [pltpu.VMEM((CHUNK,), jnp.int32) for _ in range(3)]
        + [pltpu.VMEM((CHUNK, D), jnp.float32) for _ in range(2)]
        + [
            pltpu.VMEM_SHARED((N, D), jnp.float32),
            pltpu.SemaphoreType.DMA,
            pltpu.SemaphoreType.DMA,
            pltpu.SemaphoreType.DMA,
            pltpu.SemaphoreType.DMA,
        ]
    ),
)
def _prop_call(hp_hbm, src_hbm, dst_hbm, z_hbm, out_hbm, *scr):
    idx_s = list(scr[0:3])
    idx_d = list(scr[3:6])
    rows_v = list(scr[6:8])
    out_sh, sem_i, sem_g, sem_s0, sem_s1 = scr[8:]
    _prop_body(hp_hbm, src_hbm, dst_hbm, z_hbm, out_hbm,
               idx_s, idx_d, rows_v, out_sh, sem_i, sem_g,
               [sem_s0, sem_s1])


def _dinv(d0_ref, d1_ref):
    deg = d0_ref[:, 0:1] + d1_ref[:, 0:1] + 1.0
    return lax.rsqrt(deg)


def _tca_body(x_ref, w_ref, d0_ref, d1_ref, o_ref):
    dinv = _dinv(d0_ref, d1_ref)
    o_ref[...] = jnp.dot(x_ref[...], w_ref[...],
                         preferred_element_type=jnp.float32) * dinv


def _bn(s0_ref, s1_ref, hp_ref, dinv, g_ref, bt_ref):
    u = (s0_ref[...] + s1_ref[...] + hp_ref[...]) * dinv
    m = jnp.mean(u, axis=0, keepdims=True)
    d = u - m
    v = jnp.mean(d * d, axis=0, keepdims=True)
    return d * lax.rsqrt(v + 1e-5) * g_ref[...] + bt_ref[...]


def _tcb_body(s0_ref, s1_ref, hp_ref, d0_ref, d1_ref, g_ref, bt_ref, wn_ref,
              o_ref):
    dinv = _dinv(d0_ref, d1_ref)
    y = _bn(s0_ref, s1_ref, hp_ref, dinv, g_ref, bt_ref)
    o_ref[...] = jnp.dot(y, wn_ref[...],
                         preferred_element_type=jnp.float32) * dinv


def _tcc_body(s0_ref, s1_ref, hp_ref, d0_ref, d1_ref, g_ref, bt_ref, o_ref):
    dinv = _dinv(d0_ref, d1_ref)
    y = _bn(s0_ref, s1_ref, hp_ref, dinv, g_ref, bt_ref)
    o_ref[...] = y[:1024, :]


_tca = pl.pallas_call(
    _tca_body, out_shape=jax.ShapeDtypeStruct((N, D), jnp.float32))
_tcb = pl.pallas_call(
    _tcb_body, out_shape=jax.ShapeDtypeStruct((N, D), jnp.float32))
_tcc = pl.pallas_call(
    _tcc_body, out_shape=jax.ShapeDtypeStruct((1024, D), jnp.float32))


def kernel(x, edge_index, W1, b1, W2, b2, W3, b3, g1, bt1, g2, bt2, g3, bt3):
    src = edge_index[0]
    dst = edge_index[1]
    ones_blk = jnp.ones((CHUNK, D), jnp.float32)
    z128 = jnp.zeros((ROWS, D), jnp.float32)

    deg = _deg_call(dst, ones_blk, z128)
    d0 = deg[0]
    d1 = deg[1]

    hp1 = _tca(x, W1, d0, d1)
    s1 = _prop_call(hp1, src, dst, z128)
    hp2 = _tcb(s1[0], s1[1], hp1, d0, d1,
               g1.reshape(1, D), bt1.reshape(1, D), W2)
    s2 = _prop_call(hp2, src, dst, z128)
    hp3 = _tcb(s2[0], s2[1], hp2, d0, d1,
               g2.reshape(1, D), bt2.reshape(1, D), W3)
    s3 = _prop_call(hp3, src, dst, z128)
    return _tcc(s3[0], s3[1], hp3, d0, d1,
                g3.reshape(1, D), bt3.reshape(1, D))


# speedup vs baseline: 1.6738x; 1.0047x over previous
import functools

import jax
import jax.numpy as jnp
from jax import lax
from jax.experimental import pallas as pl
from jax.experimental.pallas import tpu as pltpu
from jax.experimental.pallas import tpu_sc as plsc

N = 10000
E = 320000
D = 128
CHUNK = 128
NCHUNKS = E // CHUNK
NTILES = 32
ITERS = -(-NCHUNKS // NTILES)
PIPE_ROUNDS = -(-(ITERS + 2) // 6)
ROWS = 624
TAIL = N - 16 * ROWS

_MESH = plsc.VectorSubcoreMesh(core_axis_name="c", subcore_axis_name="s")


def _deg_body(dst_hbm, ones_hbm, z_hbm, deg_hbm, idx_v, ones_v, deg_sh,
              sem_i, sem_s):
    c = lax.axis_index("c")
    s = lax.axis_index("s")
    wid = c * 16 + s
    off = pl.multiple_of(s * ROWS, 8)
    pltpu.sync_copy(ones_hbm, ones_v)
    pltpu.sync_copy(z_hbm, deg_sh.at[pl.ds(off, ROWS)])

    @pl.when(s == 0)
    def _():
        pltpu.sync_copy(z_hbm.at[pl.ds(0, TAIL)],
                        deg_sh.at[pl.ds(16 * ROWS, TAIL)])

    plsc.subcore_barrier()

    def valid(k):
        return (wid + k * NTILES) < NCHUNKS

    def i_desc(k, ib):
        base = pl.multiple_of((wid + k * NTILES) * CHUNK, CHUNK)
        return pltpu.make_async_copy(dst_hbm.at[pl.ds(base, CHUNK)],
                                     idx_v[ib], sem_i)

    def s_desc(sb, ib):
        return pltpu.make_async_copy(ones_v, deg_sh.at[idx_v[ib]],
                                     sem_s[sb])

    i_desc(0, 0).start()

    def body(r, carry):
        for b in range(6):
            j = r * 6 + b

            @pl.when(valid(j))
            def _(j=j, b=b):
                i_desc(j, b % 3).wait()

            pm2 = valid(j - 2) if b >= 2 else ((r >= 1) & valid(j - 2))

            @pl.when(pm2)
            def _(j=j, b=b):
                s_desc((b - 2) % 2, (b - 2) % 3).wait()

            @pl.when(valid(j))
            def _(j=j, b=b):
                s_desc(b % 2, b % 3).start(add=True)

            @pl.when(valid(j + 1))
            def _(j=j, b=b):
                i_desc(j + 1, (b + 1) % 3).start()

        return carry

    lax.fori_loop(0, PIPE_ROUNDS, body, 0)
    plsc.subcore_barrier()
    pltpu.sync_copy(deg_sh.at[pl.ds(off, ROWS)],
                    deg_hbm.at[c, pl.ds(off, ROWS)])

    @pl.when(s == 0)
    def _():
        pltpu.sync_copy(deg_sh.at[pl.ds(16 * ROWS, TAIL)],
                        deg_hbm.at[c, pl.ds(16 * ROWS, TAIL)])


@functools.partial(
    pl.kernel,
    out_type=jax.ShapeDtypeStruct((2, N, D), jnp.float32),
    mesh=_MESH,
    scratch_types=[
        pltpu.VMEM((CHUNK,), jnp.int32),
        pltpu.VMEM((CHUNK,), jnp.int32),
        pltpu.VMEM((CHUNK,), jnp.int32),
        pltpu.VMEM((CHUNK, D), jnp.float32),
        pltpu.VMEM_SHARED((N, D), jnp.float32),
        pltpu.SemaphoreType.DMA,
        pltpu.SemaphoreType.DMA,
        pltpu.SemaphoreType.DMA,
    ],
)
def _deg_call(dst_hbm, ones_hbm, z_hbm, deg_hbm, i0, i1, i2, ones_v, deg_sh,
              sem_i, sem_s0, sem_s1):
    _deg_body(dst_hbm, ones_hbm, z_hbm, deg_hbm, [i0, i1, i2], ones_v,
              deg_sh, sem_i, [sem_s0, sem_s1])


def _prop_body(hp_hbm, src_hbm, dst_hbm, z_hbm, out_hbm,
               idx_s, idx_d, rows_v, out_sh, sem_i, sem_g, sem_s):
    c = lax.axis_index("c")
    s = lax.axis_index("s")
    wid = c * 16 + s
    off = pl.multiple_of(s * ROWS, 8)
    pltpu.sync_copy(z_hbm, out_sh.at[pl.ds(off, ROWS)])

    @pl.when(s == 0)
    def _():
        pltpu.sync_copy(z_hbm.at[pl.ds(0, TAIL)],
                        out_sh.at[pl.ds(16 * ROWS, TAIL)])

    plsc.subcore_barrier()

    def valid(k):
        return (wid + k * NTILES) < NCHUNKS

    def idx_descs(k, ib):
        base = pl.multiple_of((wid + k * NTILES) * CHUNK, CHUNK)
        return (
            pltpu.make_async_copy(src_hbm.at[pl.ds(base, CHUNK)],
                                  idx_s[ib], sem_i),
            pltpu.make_async_copy(dst_hbm.at[pl.ds(base, CHUNK)],
                                  idx_d[ib], sem_i),
        )

    def g_desc(rb, ib):
        return pltpu.make_async_copy(hp_hbm.at[idx_s[ib]], rows_v[rb], sem_g)

    def s_desc(rb, ib):
        return pltpu.make_async_copy(rows_v[rb], out_sh.at[idx_d[ib]],
                                     sem_s[rb])

    for d in idx_descs(0, 0):
        d.start()

    def body(r, carry):
        for b in range(6):
            j = r * 6 + b

            @pl.when(valid(j))
            def _(j=j, b=b):
                for d in idx_descs(j, b % 3):
                    d.wait()

            pm1 = valid(j - 1) if b >= 1 else ((r >= 1) & valid(j - 1))

            @pl.when(pm1)
            def _(j=j, b=b):
                g_desc((b - 1) % 2, (b - 1) % 3).wait()
                s_desc((b - 1) % 2, (b - 1) % 3).start(add=True)

            pm2 = valid(j - 2) if b >= 2 else ((r >= 1) & valid(j - 2))

            @pl.when(pm2)
            def _(j=j, b=b):
                s_desc((b - 2) % 2, (b - 2) % 3).wait()

            @pl.when(valid(j))
            def _(j=j, b=b):
                g_desc(b % 2, b % 3).start()

            @pl.when(valid(j + 1))
            def _(j=j, b=b):
                for d in idx_descs(j + 1, (b + 1) % 3):
                    d.start()

        return carry

    lax.fori_loop(0, PIPE_ROUNDS, body, 0)
    plsc.subcore_barrier()
    pltpu.sync_copy(out_sh.at[pl.ds(off, ROWS)],
                    out_hbm.at[c, pl.ds(off, ROWS)])

    @pl.when(s == 0)
    def _():
        pltpu.sync_copy(out_sh.at[pl.ds(16 * ROWS, TAIL)],
                        out_hbm.at[c, pl.ds(16 * ROWS, TAIL)])


@functools.partial(
    pl.kernel,
    out_type=jax.ShapeDtypeStruct((2, N, D), jnp.float32),
    mesh=_MESH,
    scratch_types=(
        [pltpu.VMEM((CHUNK,), jnp.int32) for _ in range(3)]
        + [pltpu.VMEM((CHUNK,), jnp.int32) for _ in range(3)]
        + [pltpu.VMEM((CHUNK, D), jnp.float32) for _ in range(2)]
        + [
            pltpu.VMEM_SHARED((N, D), jnp.float32),
            pltpu.SemaphoreType.DMA,
            pltpu.SemaphoreType.DMA,
            pltpu.SemaphoreType.DMA,
            pltpu.SemaphoreType.DMA,
        ]
    ),
)
def _prop_call(hp_hbm, src_hbm, dst_hbm, z_hbm, out_hbm, *scr):
    idx_s = list(scr[0:3])
    idx_d = list(scr[3:6])
    rows_v = list(scr[6:8])
    out_sh, sem_i, sem_g, sem_s0, sem_s1 = scr[8:]
    _prop_body(hp_hbm, src_hbm, dst_hbm, z_hbm, out_hbm,
               idx_s, idx_d, rows_v, out_sh, sem_i, sem_g,
               [sem_s0, sem_s1])


def _tcmm_body(x_ref, w_ref, o_ref):
    o_ref[...] = jnp.dot(x_ref[...], w_ref[...],
                         preferred_element_type=jnp.float32)


def _tca_body(h_ref, d0_ref, d1_ref, hp_ref, dinv_ref):
    dinv = lax.rsqrt(d0_ref[:, 0:1] + d1_ref[:, 0:1] + 1.0)
    dinv_ref[...] = dinv
    hp_ref[...] = h_ref[...] * dinv


def _bn(s0_ref, s1_ref, hp_ref, dinv, g_ref, bt_ref):
    u = (s0_ref[...] + s1_ref[...] + hp_ref[...]) * dinv
    m = jnp.mean(u, axis=0, keepdims=True)
    d = u - m
    v = jnp.mean(d * d, axis=0, keepdims=True)
    return d * lax.rsqrt(v + 1e-5) * g_ref[...] + bt_ref[...]


def _tcb_body(s0_ref, s1_ref, hp_ref, dinv_ref, g_ref, bt_ref, wn_ref,
              o_ref):
    dinv = dinv_ref[...]
    y = _bn(s0_ref, s1_ref, hp_ref, dinv, g_ref, bt_ref)
    o_ref[...] = jnp.dot(y, wn_ref[...],
                         preferred_element_type=jnp.float32) * dinv


def _tcc_body(s0_ref, s1_ref, hp_ref, dinv_ref, g_ref, bt_ref, o_ref):
    dinv = dinv_ref[...]
    y = _bn(s0_ref, s1_ref, hp_ref, dinv, g_ref, bt_ref)
    o_ref[...] = y[:1024, :]


_tcmm = pl.pallas_call(
    _tcmm_body, out_shape=jax.ShapeDtypeStruct((N, D), jnp.float32))
_tca = pl.pallas_call(
    _tca_body, out_shape=(jax.ShapeDtypeStruct((N, D), jnp.float32),
                          jax.ShapeDtypeStruct((N, 1), jnp.float32)))
_tcb = pl.pallas_call(
    _tcb_body, out_shape=jax.ShapeDtypeStruct((N, D), jnp.float32))
_tcc = pl.pallas_call(
    _tcc_body, out_shape=jax.ShapeDtypeStruct((1024, D), jnp.float32))


def kernel(x, edge_index, W1, b1, W2, b2, W3, b3, g1, bt1, g2, bt2, g3, bt3):
    src = edge_index[0]
    dst = edge_index[1]
    ones_blk = jnp.ones((CHUNK, D), jnp.float32)
    z128 = jnp.zeros((ROWS, D), jnp.float32)

    h1 = _tcmm(x, W1)
    deg = _deg_call(dst, ones_blk, z128)

    hp1, dinv = _tca(h1, deg[0], deg[1])
    s1 = _prop_call(hp1, src, dst, z128)
    hp2 = _tcb(s1[0], s1[1], hp1, dinv,
               g1.reshape(1, D), bt1.reshape(1, D), W2)
    s2 = _prop_call(hp2, src, dst, z128)
    hp3 = _tcb(s2[0], s2[1], hp2, dinv,
               g2.reshape(1, D), bt2.reshape(1, D), W3)
    s3 = _prop_call(hp3, src, dst, z128)
    return _tcc(s3[0], s3[1], hp3, dinv,
                g3.reshape(1, D), bt3.reshape(1, D))
